# Initial kernel scaffold; baseline (speedup 1.0000x reference)
import jax
import jax.numpy as jnp
from jax.experimental import pallas as pl


def kernel(x, pos, edge_attr, edge_index, pW, pb, fW, fb, tW, tb, mW1, mb1, mW2, mb2, eW1, eb1, eW2, eb2):
    raise NotImplementedError("write your pallas kernel here")


# speedup vs baseline: 4.1785x; 4.1785x over previous
import functools

import jax
import jax.numpy as jnp
from jax import lax
from jax.experimental import pallas as pl
from jax.experimental.pallas import tpu as pltpu
from jax.experimental.pallas import tpu_sc as plsc

N = 50000
E = 800000
NCAT = 16
SIGMA = 32
DISTE = 32
LIG_E = 4
H = 16

NC = 2
NS = 16
NW = NC * NS
EPW = E // NW
CHUNK = 1000
NCHUNK = EPW // CHUNK
G = 125
NG = CHUNK // G
CPAD = 1008
NPAD = 50048
RPT = NPAD // NS

_mesh = plsc.VectorSubcoreMesh(
    core_axis_name="c", subcore_axis_name="s", num_cores=NC, num_subcores=NS)


def _node_mlp_body(x_ref, bd_ref, bcat_ref, mW1_ref, mb1_ref, mW2_ref,
                   mb2_ref, h_ref):
    w0 = bd_ref[...] @ mW1_ref[...]
    b0 = bcat_ref[...] @ mW1_ref[...] + mb1_ref[...]
    h1 = jnp.maximum(x_ref[...] @ w0 + b0, 0.0)
    h_ref[...] = h1 @ mW2_ref[...] + mb2_ref[...]


def _node_mlp(x, bd, bcat, mW1, mb1, mW2, mb2):
    bn = 2000
    grid = N // bn
    return pl.pallas_call(
        _node_mlp_body,
        grid=(grid,),
        in_specs=[
            pl.BlockSpec((bn, 51), lambda i: (i, 0)),
            pl.BlockSpec((51, 24), lambda i: (0, 0)),
            pl.BlockSpec((1, 24), lambda i: (0, 0)),
            pl.BlockSpec((24, H), lambda i: (0, 0)),
            pl.BlockSpec((1, H), lambda i: (0, 0)),
            pl.BlockSpec((H, H), lambda i: (0, 0)),
            pl.BlockSpec((1, H), lambda i: (0, 0)),
        ],
        out_specs=pl.BlockSpec((bn, H), lambda i: (i, 0)),
        out_shape=jax.ShapeDtypeStruct((N, H), jnp.float32),
    )(x, bd, bcat, mW1, mb1, mW2, mb2)


_sc_params = pltpu.CompilerParams(use_tc_tiling_on_sc=False)


@functools.partial(
    pl.kernel,
    out_type=jax.ShapeDtypeStruct((E, 16), jnp.float32),
    mesh=_mesh,
    compiler_params=_sc_params,
    scratch_types=[
        pltpu.VMEM((NG, G), jnp.int32),
        pltpu.VMEM((NG, G), jnp.int32),
        pltpu.VMEM((CHUNK, 16), jnp.float32),
        pltpu.VMEM((CHUNK, 16), jnp.float32),
        pltpu.SemaphoreType.DMA,
    ],
)
def _dist_kernel(posp_hbm, src_hbm, dst_hbm, sq_hbm, sidx, didx, ps, pd, sem):
    c = lax.axis_index("c")
    s = lax.axis_index("s")
    wid = c * NS + s

    def chunk_body(k, carry):
        cid = wid * NCHUNK + k
        base = cid * CHUNK
        base_rows = cid * NG
        pltpu.sync_copy(src_hbm.at[pl.ds(base_rows, NG)], sidx)
        pltpu.sync_copy(dst_hbm.at[pl.ds(base_rows, NG)], didx)
        descs = []
        for g in range(NG):
            descs.append(pltpu.async_copy(
                posp_hbm.at[sidx.at[g]], ps.at[pl.ds(g * G, G)], sem))
            descs.append(pltpu.async_copy(
                posp_hbm.at[didx.at[g]], pd.at[pl.ds(g * G, G)], sem))
        for d in descs:
            d.wait()

        def sqd(i, carry2):
            v = ps[i] - pd[i]
            ps[i] = v * v
            return carry2

        lax.fori_loop(0, CHUNK, sqd, 0)
        pltpu.sync_copy(ps, sq_hbm.at[pl.ds(base, CHUNK)])
        return carry

    lax.fori_loop(0, NCHUNK, chunk_body, 0)


def _edge_mlp_body(ea_ref, sq_ref, eW1a_ref, eW1b_ref, eb1_ref, eW2_ref,
                   eb2_ref, e_ref):
    delta = 5.0 / (DISTE - 1)
    coeff = -0.5 / (delta * delta)
    offs = lax.broadcasted_iota(jnp.int32, (1, DISTE), 1).astype(jnp.float32) * delta
    d2 = jnp.sum(sq_ref[...], axis=1, keepdims=True)
    d = jnp.sqrt(d2 + 1e-12)
    dist_exp = jnp.exp(coeff * (d - offs) ** 2)
    z = ea_ref[...] @ eW1a_ref[...] + dist_exp @ eW1b_ref[...] + eb1_ref[...]
    e_ref[...] = jnp.maximum(z, 0.0) @ eW2_ref[...] + eb2_ref[...]


def _edge_mlp(edge_attr, sq, eW1a, eW1b, eb1, eW2, eb2):
    be = 4000
    grid = E // be
    return pl.pallas_call(
        _edge_mlp_body,
        grid=(grid,),
        in_specs=[
            pl.BlockSpec((be, LIG_E + SIGMA), lambda i: (i, 0)),
            pl.BlockSpec((be, 16), lambda i: (i, 0)),
            pl.BlockSpec((LIG_E + SIGMA, H), lambda i: (0, 0)),
            pl.BlockSpec((DISTE, H), lambda i: (0, 0)),
            pl.BlockSpec((1, H), lambda i: (0, 0)),
            pl.BlockSpec((H, H), lambda i: (0, 0)),
            pl.BlockSpec((1, H), lambda i: (0, 0)),
        ],
        out_specs=pl.BlockSpec((be, H), lambda i: (i, 0)),
        out_shape=jax.ShapeDtypeStruct((E, H), jnp.float32),
    )(edge_attr, sq, eW1a, eW1b, eb1, eW2, eb2)


@functools.partial(
    pl.kernel,
    out_type=jax.ShapeDtypeStruct((NC, NPAD, H), jnp.float32),
    mesh=_mesh,
    compiler_params=_sc_params,
    scratch_types=[
        pltpu.VMEM((NG, G), jnp.int32),
        pltpu.VMEM((NG, G), jnp.int32),
        pltpu.VMEM((CHUNK, H), jnp.float32),
        pltpu.VMEM((CHUNK, H), jnp.float32),
        pltpu.VMEM_SHARED((NPAD, H), jnp.float32),
        pltpu.SemaphoreType.DMA,
    ],
)
def _scatter_kernel(h_hbm, e_hbm, src_hbm, dst_hbm, zeros_hbm, out_hbm,
                    sidx, didx, eb, hb, accum, sem):
    c = lax.axis_index("c")
    s = lax.axis_index("s")
    wid = c * NS + s
    pltpu.sync_copy(zeros_hbm, accum.at[pl.ds(s * RPT, RPT)])
    plsc.subcore_barrier()

    def chunk_body(k, carry):
        cid = wid * NCHUNK + k
        base = cid * CHUNK
        base_rows = cid * NG
        pltpu.sync_copy(src_hbm.at[pl.ds(base_rows, NG)], sidx)
        pltpu.sync_copy(dst_hbm.at[pl.ds(base_rows, NG)], didx)
        pltpu.sync_copy(e_hbm.at[pl.ds(base, CHUNK)], eb)
        descs = [
            pltpu.async_copy(h_hbm.at[sidx.at[g]], hb.at[pl.ds(g * G, G)], sem)
            for g in range(NG)
        ]
        for d in descs:
            d.wait()

        def mul(i, carry2):
            hb[i] = hb[i] * eb[i]
            return carry2

        lax.fori_loop(0, CHUNK, mul, 0)
        for g in range(NG):
            pltpu.sync_copy(hb.at[pl.ds(g * G, G)], accum.at[didx.at[g]],
                            add=True)
        return carry

    lax.fori_loop(0, NCHUNK, chunk_body, 0)
    plsc.subcore_barrier()
    pltpu.sync_copy(accum.at[pl.ds(s * RPT, RPT)],
                    out_hbm.at[c, pl.ds(s * RPT, RPT)])


def _final_body(h_ref, a0_ref, a1_ref, o_ref):
    o_ref[...] = h_ref[...] + (a0_ref[...] + a1_ref[...]) * (1.0 / 1000.0)


def _final_add(h, a0, a1):
    bn = 2000
    grid = N // bn
    return pl.pallas_call(
        _final_body,
        grid=(grid,),
        in_specs=[
            pl.BlockSpec((bn, H), lambda i: (i, 0)),
            pl.BlockSpec((bn, H), lambda i: (i, 0)),
            pl.BlockSpec((bn, H), lambda i: (i, 0)),
        ],
        out_specs=pl.BlockSpec((bn, H), lambda i: (i, 0)),
        out_shape=jax.ShapeDtypeStruct((N, H), jnp.float32),
    )(h, a0, a1)


def kernel(x, pos, edge_attr, edge_index, pW, pb, fW, fb, tW, tb, mW1, mb1,
           mW2, mb2, eW1, eb1, eW2, eb2):
    posp = jnp.pad(pos, ((0, 0), (0, 16 - 3)))
    src2d = edge_index[0].reshape(E // G, G)
    dst2d = edge_index[1].reshape(E // G, G)
    bd = jax.scipy.linalg.block_diag(pW, fW, tW)
    bcat = jnp.concatenate([pb, fb, tb]).reshape(1, 24)
    zeros_rpt = jnp.zeros((RPT, H), jnp.float32)

    h = _node_mlp(x, bd, bcat, mW1, mb1.reshape(1, H), mW2, mb2.reshape(1, H))
    sq = _dist_kernel(posp, src2d, dst2d)
    e = _edge_mlp(edge_attr, sq, eW1[:LIG_E + SIGMA],
                  eW1[LIG_E + SIGMA:], eb1.reshape(1, H), eW2,
                  eb2.reshape(1, H))
    agg = _scatter_kernel(h, e, src2d, dst2d, zeros_rpt)
    return _final_add(h, agg[0, :N], agg[1, :N])


# speedup vs baseline: 6.8892x; 1.6487x over previous
import functools

import jax
import jax.numpy as jnp
from jax import lax
from jax.experimental import pallas as pl
from jax.experimental.pallas import tpu as pltpu
from jax.experimental.pallas import tpu_sc as plsc

N = 50000
E = 800000
NCAT = 16
SIGMA = 32
DISTE = 32
LIG_E = 4
H = 16

NC = 2
NS = 16
NW = NC * NS
CHUNK = 1000
NCHUNK = E // CHUNK // NW
G = 125
NG = CHUNK // G
EP8 = E // 8
TCB = 8000
PB = TCB // 8
NPAD = 50048
RPT = NPAD // NS

_mesh = plsc.VectorSubcoreMesh(
    core_axis_name="c", subcore_axis_name="s", num_cores=NC, num_subcores=NS)
_sc_params = pltpu.CompilerParams(use_tc_tiling_on_sc=False)


def _node_mlp_body(x_ref, bd_ref, bcat_ref, mW1_ref, mb1_ref, mW2_ref,
                   mb2_ref, h_ref):
    w0 = bd_ref[...] @ mW1_ref[...]
    b0 = bcat_ref[...] @ mW1_ref[...] + mb1_ref[...]
    h1 = jnp.maximum(x_ref[...] @ w0 + b0, 0.0)
    h_ref[...] = h1 @ mW2_ref[...] + mb2_ref[...]


def _node_mlp(x, bd, bcat, mW1, mb1, mW2, mb2):
    bn = 2000
    grid = N // bn
    return pl.pallas_call(
        _node_mlp_body,
        grid=(grid,),
        in_specs=[
            pl.BlockSpec((bn, 51), lambda i: (i, 0)),
            pl.BlockSpec((51, 24), lambda i: (0, 0)),
            pl.BlockSpec((1, 24), lambda i: (0, 0)),
            pl.BlockSpec((24, H), lambda i: (0, 0)),
            pl.BlockSpec((1, H), lambda i: (0, 0)),
            pl.BlockSpec((H, H), lambda i: (0, 0)),
            pl.BlockSpec((1, H), lambda i: (0, 0)),
        ],
        out_specs=pl.BlockSpec((bn, H), lambda i: (i, 0)),
        out_shape=jax.ShapeDtypeStruct((N, H), jnp.float32),
    )(x, bd, bcat, mW1, mb1, mW2, mb2)


@functools.partial(
    pl.kernel,
    out_type=jax.ShapeDtypeStruct((EP8, 128), jnp.float32),
    mesh=_mesh,
    compiler_params=_sc_params,
    scratch_types=[
        pltpu.VMEM((NG, G), jnp.int32),
        pltpu.VMEM((NG, G), jnp.int32),
        pltpu.VMEM((CHUNK, 16), jnp.float32),
        pltpu.VMEM((CHUNK, 16), jnp.float32),
        pltpu.SemaphoreType.DMA,
    ],
)
def _dist_kernel(posp_hbm, src_hbm, dst_hbm, sq_hbm, sidx, didx, ps, pd, sem):
    c = lax.axis_index("c")
    s = lax.axis_index("s")
    wid = c * NS + s

    def chunk_body(k, carry):
        cid = wid * NCHUNK + k
        base_rows = cid * NG
        pltpu.sync_copy(src_hbm.at[pl.ds(base_rows, NG)], sidx)
        pltpu.sync_copy(dst_hbm.at[pl.ds(base_rows, NG)], didx)
        descs = []
        for g in range(NG):
            descs.append(pltpu.async_copy(
                posp_hbm.at[sidx.at[g]], ps.at[pl.ds(g * G, G)], sem))
            descs.append(pltpu.async_copy(
                posp_hbm.at[didx.at[g]], pd.at[pl.ds(g * G, G)], sem))
        for d in descs:
            d.wait()

        def sqd(i, carry2):
            v = ps[i] - pd[i]
            ps[i] = v * v
            return carry2

        lax.fori_loop(0, CHUNK, sqd, 0)
        pltpu.sync_copy(
            ps,
            sq_hbm.at[pl.ds((cid // 8) * PB, CHUNK), pl.ds((cid % 8) * 16, 16)])
        return carry

    lax.fori_loop(0, NCHUNK, chunk_body, 0)


def _edge_mlp_body(ea_ref, sqp_ref, eW1a_ref, w1bd_ref, b1_ref, w2bd_ref,
                   b2_ref, e_ref):
    delta = 5.0 / (DISTE - 1)
    coeff = -0.5 / (delta * delta)
    sqp = sqp_ref[...]
    S = (lax.broadcasted_iota(jnp.int32, (128, 8), 0) // 16 ==
         lax.broadcasted_iota(jnp.int32, (128, 8), 1)).astype(jnp.float32)
    d2 = sqp @ S
    d = jnp.sqrt(d2 + 1e-12)
    R = (lax.broadcasted_iota(jnp.int32, (8, 256), 0) ==
         lax.broadcasted_iota(jnp.int32, (8, 256), 1) // DISTE
         ).astype(jnp.float32)
    dbc = d @ R
    offs = (lax.broadcasted_iota(jnp.int32, (1, 256), 1) % DISTE
            ).astype(jnp.float32) * delta
    dist = jnp.exp(coeff * (dbc - offs) ** 2)
    contrib = dist @ w1bd_ref[...]
    ea = ea_ref[...]
    w1a = eW1a_ref[...]
    zs = [ea[PB * g:PB * (g + 1), :] @ w1a for g in range(8)]
    z_ea = jnp.concatenate(zs, axis=1)
    z = jnp.maximum(z_ea + contrib + b1_ref[...], 0.0)
    e_ref[...] = z @ w2bd_ref[...] + b2_ref[...]


def _edge_mlp(edge_attr, sq_p, eW1a, w1bd, b1rep, w2bd, b2rep):
    grid = E // TCB
    return pl.pallas_call(
        _edge_mlp_body,
        grid=(grid,),
        in_specs=[
            pl.BlockSpec((TCB, LIG_E + SIGMA), lambda i: (i, 0)),
            pl.BlockSpec((PB, 128), lambda i: (i, 0)),
            pl.BlockSpec((LIG_E + SIGMA, H), lambda i: (0, 0)),
            pl.BlockSpec((8 * DISTE, 128), lambda i: (0, 0)),
            pl.BlockSpec((1, 128), lambda i: (0, 0)),
            pl.BlockSpec((128, 128), lambda i: (0, 0)),
            pl.BlockSpec((1, 128), lambda i: (0, 0)),
        ],
        out_specs=pl.BlockSpec((PB, 128), lambda i: (i, 0)),
        out_shape=jax.ShapeDtypeStruct((EP8, 128), jnp.float32),
    )(edge_attr, sq_p, eW1a, w1bd, b1rep, w2bd, b2rep)


@functools.partial(
    pl.kernel,
    out_type=jax.ShapeDtypeStruct((NC, NPAD, H), jnp.float32),
    mesh=_mesh,
    compiler_params=_sc_params,
    scratch_types=[
        pltpu.VMEM((NG, G), jnp.int32),
        pltpu.VMEM((NG, G), jnp.int32),
        pltpu.VMEM((CHUNK, H), jnp.float32),
        pltpu.VMEM((CHUNK, H), jnp.float32),
        pltpu.VMEM_SHARED((NPAD, H), jnp.float32),
        pltpu.SemaphoreType.DMA,
    ],
)
def _scatter_kernel(h_hbm, e_hbm, src_hbm, dst_hbm, zeros_hbm, out_hbm,
                    sidx, didx, eb, hb, accum, sem):
    c = lax.axis_index("c")
    s = lax.axis_index("s")
    wid = c * NS + s
    pltpu.sync_copy(zeros_hbm, accum.at[pl.ds(s * RPT, RPT)])
    plsc.subcore_barrier()

    def chunk_body(k, carry):
        cid = wid * NCHUNK + k
        base_rows = cid * NG
        pltpu.sync_copy(src_hbm.at[pl.ds(base_rows, NG)], sidx)
        pltpu.sync_copy(dst_hbm.at[pl.ds(base_rows, NG)], didx)
        pltpu.sync_copy(
            e_hbm.at[pl.ds((cid // 8) * PB, CHUNK), pl.ds((cid % 8) * 16, 16)],
            eb)
        descs = [
            pltpu.async_copy(h_hbm.at[sidx.at[g]], hb.at[pl.ds(g * G, G)], sem)
            for g in range(NG)
        ]
        for d in descs:
            d.wait()

        def mul(i, carry2):
            hb[i] = hb[i] * eb[i]
            return carry2

        lax.fori_loop(0, CHUNK, mul, 0)
        for g in range(NG):
            pltpu.sync_copy(hb.at[pl.ds(g * G, G)], accum.at[didx.at[g]],
                            add=True)
        return carry

    lax.fori_loop(0, NCHUNK, chunk_body, 0)
    plsc.subcore_barrier()
    pltpu.sync_copy(accum.at[pl.ds(s * RPT, RPT)],
                    out_hbm.at[c, pl.ds(s * RPT, RPT)])


def _final_body(h_ref, a0_ref, a1_ref, o_ref):
    o_ref[...] = h_ref[...] + (a0_ref[...] + a1_ref[...]) * (1.0 / 1000.0)


def _final_add(h, a0, a1):
    bn = 2000
    grid = N // bn
    return pl.pallas_call(
        _final_body,
        grid=(grid,),
        in_specs=[
            pl.BlockSpec((bn, H), lambda i: (i, 0)),
            pl.BlockSpec((bn, H), lambda i: (i, 0)),
            pl.BlockSpec((bn, H), lambda i: (i, 0)),
        ],
        out_specs=pl.BlockSpec((bn, H), lambda i: (i, 0)),
        out_shape=jax.ShapeDtypeStruct((N, H), jnp.float32),
    )(h, a0, a1)


def kernel(x, pos, edge_attr, edge_index, pW, pb, fW, fb, tW, tb, mW1, mb1,
           mW2, mb2, eW1, eb1, eW2, eb2):
    posp = jnp.pad(pos, ((0, 0), (0, 16 - 3)))
    src2d = edge_index[0].reshape(E // G, G)
    dst2d = edge_index[1].reshape(E // G, G)
    bd = jax.scipy.linalg.block_diag(pW, fW, tW)
    bcat = jnp.concatenate([pb, fb, tb]).reshape(1, 24)
    eW1a = eW1[:LIG_E + SIGMA]
    w1bd = jax.scipy.linalg.block_diag(*([eW1[LIG_E + SIGMA:]] * 8))
    w2bd = jax.scipy.linalg.block_diag(*([eW2] * 8))
    b1rep = jnp.tile(eb1, 8).reshape(1, 128)
    b2rep = jnp.tile(eb2, 8).reshape(1, 128)
    zeros_rpt = jnp.zeros((RPT, H), jnp.float32)

    h = _node_mlp(x, bd, bcat, mW1, mb1.reshape(1, H), mW2, mb2.reshape(1, H))
    sq_p = _dist_kernel(posp, src2d, dst2d)
    e_p = _edge_mlp(edge_attr, sq_p, eW1a, w1bd, b1rep, w2bd, b2rep)
    agg = _scatter_kernel(h, e_p, src2d, dst2d, zeros_rpt)
    return _final_add(h, agg[0, :N], agg[1, :N])


# speedup vs baseline: 7.9027x; 1.1471x over previous
import functools

import jax
import jax.numpy as jnp
from jax import lax
from jax.experimental import pallas as pl
from jax.experimental.pallas import tpu as pltpu
from jax.experimental.pallas import tpu_sc as plsc

N = 50000
E = 800000
NCAT = 16
SIGMA = 32
DISTE = 32
LIG_E = 4
H = 16

NC = 2
NS = 16
NW = NC * NS
CHUNK = 1000
NCHUNK = E // CHUNK // NW
G = 125
NG = CHUNK // G
EP8 = E // 8
TCB = 8000
PB = TCB // 8
NPAD = 50048
RPT = NPAD // NS

_mesh = plsc.VectorSubcoreMesh(
    core_axis_name="c", subcore_axis_name="s", num_cores=NC, num_subcores=NS)
_sc_params = pltpu.CompilerParams(use_tc_tiling_on_sc=False)


def _node_mlp_body(x_ref, bd_ref, bcat_ref, mW1_ref, mb1_ref, mW2_ref,
                   mb2_ref, h_ref):
    w0 = bd_ref[...] @ mW1_ref[...]
    b0 = bcat_ref[...] @ mW1_ref[...] + mb1_ref[...]
    h1 = jnp.maximum(x_ref[...] @ w0 + b0, 0.0)
    h_ref[...] = h1 @ mW2_ref[...] + mb2_ref[...]


def _node_mlp(x, bd, bcat, mW1, mb1, mW2, mb2):
    bn = 2000
    grid = N // bn
    return pl.pallas_call(
        _node_mlp_body,
        grid=(grid,),
        in_specs=[
            pl.BlockSpec((bn, 51), lambda i: (i, 0)),
            pl.BlockSpec((51, 24), lambda i: (0, 0)),
            pl.BlockSpec((1, 24), lambda i: (0, 0)),
            pl.BlockSpec((24, H), lambda i: (0, 0)),
            pl.BlockSpec((1, H), lambda i: (0, 0)),
            pl.BlockSpec((H, H), lambda i: (0, 0)),
            pl.BlockSpec((1, H), lambda i: (0, 0)),
        ],
        out_specs=pl.BlockSpec((bn, H), lambda i: (i, 0)),
        out_shape=jax.ShapeDtypeStruct((N, H), jnp.float32),
    )(x, bd, bcat, mW1, mb1, mW2, mb2)


@functools.partial(
    pl.kernel,
    out_type=jax.ShapeDtypeStruct((EP8, 128), jnp.float32),
    mesh=_mesh,
    compiler_params=_sc_params,
    scratch_types=[
        pltpu.VMEM((2, NG, G), jnp.int32),
        pltpu.VMEM((2, NG, G), jnp.int32),
        pltpu.VMEM((2, CHUNK, 16), jnp.float32),
        pltpu.VMEM((2, CHUNK, 16), jnp.float32),
        pltpu.SemaphoreType.DMA,
        pltpu.SemaphoreType.DMA,
        pltpu.SemaphoreType.DMA,
        pltpu.SemaphoreType.DMA,
        pltpu.SemaphoreType.DMA,
        pltpu.SemaphoreType.DMA,
    ],
)
def _dist_kernel(posp_hbm, src_hbm, dst_hbm, sq_hbm, sidx, didx, ps, pd,
                 isem0, isem1, gsem0, gsem1, wsem0, wsem1):
    c = lax.axis_index("c")
    s = lax.axis_index("s")
    wid = c * NS + s
    isem = (isem0, isem1)
    gsem = (gsem0, gsem1)
    wsem = (wsem0, wsem1)

    def fire_idx(k):
        sl = k % 2
        base_rows = (wid * NCHUNK + k) * NG
        return [
            pltpu.async_copy(src_hbm.at[pl.ds(base_rows, NG)], sidx.at[sl],
                             isem[sl]),
            pltpu.async_copy(dst_hbm.at[pl.ds(base_rows, NG)], didx.at[sl],
                             isem[sl]),
        ]

    def fire_gather(k):
        sl = k % 2
        ds_ = []
        for g in range(NG):
            ds_.append(pltpu.async_copy(
                posp_hbm.at[sidx.at[sl, g]], ps.at[sl, pl.ds(g * G, G)],
                gsem[sl]))
            ds_.append(pltpu.async_copy(
                posp_hbm.at[didx.at[sl, g]], pd.at[sl, pl.ds(g * G, G)],
                gsem[sl]))
        return ds_

    def fire_write(k):
        sl = k % 2
        cid = wid * NCHUNK + k
        return [pltpu.async_copy(
            ps.at[sl],
            sq_hbm.at[pl.ds((cid // 8) * PB, CHUNK),
                      pl.ds((cid % 8) * 16, 16)],
            wsem[sl])]

    idx_d = {0: fire_idx(0), 1: fire_idx(1)}
    for d in idx_d[0]:
        d.wait()
    gat_d = {0: fire_gather(0)}
    wr_d = {}
    for k in range(NCHUNK):
        sl = k % 2
        for d in gat_d[k]:
            d.wait()
        if k + 2 < NCHUNK:
            idx_d[k + 2] = fire_idx(k + 2)
        if k - 1 in wr_d:
            for d in wr_d[k - 1]:
                d.wait()
        if k + 1 < NCHUNK:
            for d in idx_d[k + 1]:
                d.wait()
            gat_d[k + 1] = fire_gather(k + 1)

        def sqd(i, carry2):
            v = ps[sl, i] - pd[sl, i]
            ps[sl, i] = v * v
            return carry2

        lax.fori_loop(0, CHUNK, sqd, 0)
        wr_d[k] = fire_write(k)
    for d in wr_d[NCHUNK - 1]:
        d.wait()


def _edge_mlp_body(ea_ref, sqp_ref, eW1a_ref, w1bd_ref, b1_ref, w2bd_ref,
                   b2_ref, e_ref):
    delta = 5.0 / (DISTE - 1)
    coeff = -0.5 / (delta * delta)
    sqp = sqp_ref[...]
    S = (lax.broadcasted_iota(jnp.int32, (128, 8), 0) // 16 ==
         lax.broadcasted_iota(jnp.int32, (128, 8), 1)).astype(jnp.float32)
    d2 = sqp @ S
    d = jnp.sqrt(d2 + 1e-12)
    R = (lax.broadcasted_iota(jnp.int32, (8, 256), 0) ==
         lax.broadcasted_iota(jnp.int32, (8, 256), 1) // DISTE
         ).astype(jnp.float32)
    dbc = d @ R
    offs = (lax.broadcasted_iota(jnp.int32, (1, 256), 1) % DISTE
            ).astype(jnp.float32) * delta
    dist = jnp.exp(coeff * (dbc - offs) ** 2)
    contrib = dist @ w1bd_ref[...]
    ea = ea_ref[...]
    w1a = eW1a_ref[...]
    zs = [ea[PB * g:PB * (g + 1), :] @ w1a for g in range(8)]
    z_ea = jnp.concatenate(zs, axis=1)
    z = jnp.maximum(z_ea + contrib + b1_ref[...], 0.0)
    e_ref[...] = z @ w2bd_ref[...] + b2_ref[...]


def _edge_mlp(edge_attr, sq_p, eW1a, w1bd, b1rep, w2bd, b2rep):
    grid = E // TCB
    return pl.pallas_call(
        _edge_mlp_body,
        grid=(grid,),
        in_specs=[
            pl.BlockSpec((TCB, LIG_E + SIGMA), lambda i: (i, 0)),
            pl.BlockSpec((PB, 128), lambda i: (i, 0)),
            pl.BlockSpec((LIG_E + SIGMA, H), lambda i: (0, 0)),
            pl.BlockSpec((8 * DISTE, 128), lambda i: (0, 0)),
            pl.BlockSpec((1, 128), lambda i: (0, 0)),
            pl.BlockSpec((128, 128), lambda i: (0, 0)),
            pl.BlockSpec((1, 128), lambda i: (0, 0)),
        ],
        out_specs=pl.BlockSpec((PB, 128), lambda i: (i, 0)),
        out_shape=jax.ShapeDtypeStruct((EP8, 128), jnp.float32),
    )(edge_attr, sq_p, eW1a, w1bd, b1rep, w2bd, b2rep)


@functools.partial(
    pl.kernel,
    out_type=jax.ShapeDtypeStruct((NC, NPAD, H), jnp.float32),
    mesh=_mesh,
    compiler_params=_sc_params,
    scratch_types=[
        pltpu.VMEM((2, NG, G), jnp.int32),
        pltpu.VMEM((2, NG, G), jnp.int32),
        pltpu.VMEM((2, CHUNK, H), jnp.float32),
        pltpu.VMEM((2, CHUNK, H), jnp.float32),
        pltpu.VMEM_SHARED((NPAD, H), jnp.float32),
        pltpu.SemaphoreType.DMA,
        pltpu.SemaphoreType.DMA,
        pltpu.SemaphoreType.DMA,
        pltpu.SemaphoreType.DMA,
        pltpu.SemaphoreType.DMA,
        pltpu.SemaphoreType.DMA,
    ],
)
def _scatter_kernel(h_hbm, e_hbm, src_hbm, dst_hbm, zeros_hbm, out_hbm,
                    sidx, didx, eb, hb, accum,
                    isem0, isem1, gsem0, gsem1, ssem0, ssem1):
    c = lax.axis_index("c")
    s = lax.axis_index("s")
    wid = c * NS + s
    isem = (isem0, isem1)
    gsem = (gsem0, gsem1)
    ssem = (ssem0, ssem1)

    def fire_loads(k):
        sl = k % 2
        cid = wid * NCHUNK + k
        base_rows = cid * NG
        return [
            pltpu.async_copy(src_hbm.at[pl.ds(base_rows, NG)], sidx.at[sl],
                             isem[sl]),
            pltpu.async_copy(dst_hbm.at[pl.ds(base_rows, NG)], didx.at[sl],
                             isem[sl]),
            pltpu.async_copy(
                e_hbm.at[pl.ds((cid // 8) * PB, CHUNK),
                         pl.ds((cid % 8) * 16, 16)],
                eb.at[sl], isem[sl]),
        ]

    def fire_gather(k):
        sl = k % 2
        return [
            pltpu.async_copy(h_hbm.at[sidx.at[sl, g]],
                             hb.at[sl, pl.ds(g * G, G)], gsem[sl])
            for g in range(NG)
        ]

    def fire_scatter(k):
        sl = k % 2
        return [
            pltpu.async_copy(hb.at[sl, pl.ds(g * G, G)],
                             accum.at[didx.at[sl, g]], ssem[sl], add=True)
            for g in range(NG)
        ]

    ld_d = {0: fire_loads(0)}
    pltpu.sync_copy(zeros_hbm, accum.at[pl.ds(s * RPT, RPT)])
    plsc.subcore_barrier()
    for d in ld_d[0]:
        d.wait()
    gat_d = {0: fire_gather(0)}
    sc_d = {}
    for k in range(NCHUNK):
        sl = k % 2
        for d in gat_d[k]:
            d.wait()
        if k - 1 in sc_d:
            for d in sc_d[k - 1]:
                d.wait()
        if k + 1 < NCHUNK:
            ld_d[k + 1] = fire_loads(k + 1)

        def mul(i, carry2):
            hb[sl, i] = hb[sl, i] * eb[sl, i]
            return carry2

        lax.fori_loop(0, CHUNK, mul, 0)
        if k + 1 < NCHUNK:
            for d in ld_d[k + 1]:
                d.wait()
            gat_d[k + 1] = fire_gather(k + 1)
        sc_d[k] = fire_scatter(k)
    for d in sc_d[NCHUNK - 1]:
        d.wait()
    plsc.subcore_barrier()
    pltpu.sync_copy(accum.at[pl.ds(s * RPT, RPT)],
                    out_hbm.at[c, pl.ds(s * RPT, RPT)])


NROW = NPAD // NW
NLAST = N - (NW - 1) * NROW


@functools.partial(
    pl.kernel,
    out_type=jax.ShapeDtypeStruct((N, H), jnp.float32),
    mesh=_mesh,
    compiler_params=_sc_params,
    scratch_types=[
        pltpu.VMEM((NROW, H), jnp.float32),
        pltpu.VMEM((NROW, H), jnp.float32),
        pltpu.VMEM((NROW, H), jnp.float32),
        pltpu.SemaphoreType.DMA,
    ],
)
def _final_kernel(h_hbm, agg_hbm, o_hbm, hb, a0, a1, sem):
    c = lax.axis_index("c")
    s = lax.axis_index("s")
    wid = c * NS + s
    r0 = wid * NROW

    def doit(nrow):
        ds_ = [
            pltpu.async_copy(h_hbm.at[pl.ds(r0, nrow)],
                             hb.at[pl.ds(0, nrow)], sem),
            pltpu.async_copy(agg_hbm.at[0, pl.ds(r0, nrow)],
                             a0.at[pl.ds(0, nrow)], sem),
            pltpu.async_copy(agg_hbm.at[1, pl.ds(r0, nrow)],
                             a1.at[pl.ds(0, nrow)], sem),
        ]
        for d in ds_:
            d.wait()

        def add(i, cy):
            hb[i] = hb[i] + (a0[i] + a1[i])
            return cy

        lax.fori_loop(0, nrow, add, 0)
        pltpu.sync_copy(hb.at[pl.ds(0, nrow)], o_hbm.at[pl.ds(r0, nrow)])

    @pl.when(wid < NW - 1)
    def _():
        doit(NROW)

    @pl.when(wid == NW - 1)
    def _():
        doit(NLAST)


def kernel(x, pos, edge_attr, edge_index, pW, pb, fW, fb, tW, tb, mW1, mb1,
           mW2, mb2, eW1, eb1, eW2, eb2):
    posp = jnp.pad(pos, ((0, 0), (0, 16 - 3)))
    src2d = edge_index[0].reshape(E // G, G)
    dst2d = edge_index[1].reshape(E // G, G)
    bd = jax.scipy.linalg.block_diag(pW, fW, tW)
    bcat = jnp.concatenate([pb, fb, tb]).reshape(1, 24)
    eW1a = eW1[:LIG_E + SIGMA]
    w1bd = jax.scipy.linalg.block_diag(*([eW1[LIG_E + SIGMA:]] * 8))
    w2bd = jax.scipy.linalg.block_diag(*([eW2] * 8)) * (1.0 / 1000.0)
    b1rep = jnp.tile(eb1, 8).reshape(1, 128)
    b2rep = jnp.tile(eb2, 8).reshape(1, 128) * (1.0 / 1000.0)
    zeros_rpt = jnp.zeros((RPT, H), jnp.float32)

    h = _node_mlp(x, bd, bcat, mW1, mb1.reshape(1, H), mW2, mb2.reshape(1, H))
    sq_p = _dist_kernel(posp, src2d, dst2d)
    e_p = _edge_mlp(edge_attr, sq_p, eW1a, w1bd, b1rep, w2bd, b2rep)
    agg = _scatter_kernel(h, e_p, src2d, dst2d, zeros_rpt)
    return _final_kernel(h, agg)


# speedup vs baseline: 10.2799x; 1.3008x over previous
import functools

import jax
import jax.numpy as jnp
from jax import lax
from jax.experimental import pallas as pl
from jax.experimental.pallas import tpu as pltpu
from jax.experimental.pallas import tpu_sc as plsc

N = 50000
E = 800000
NCAT = 16
SIGMA = 32
DISTE = 32
LIG_E = 4
H = 16

NC = 2
NS = 16
NW = NC * NS
CHUNK = 1000
NCHUNK = E // CHUNK // NW
G = 125
NG = CHUNK // G
EP8 = E // 8
TCB = 16000
PB = TCB // 8
NPAD = 50048
RPT = NPAD // NS

_mesh = plsc.VectorSubcoreMesh(
    core_axis_name="c", subcore_axis_name="s", num_cores=NC, num_subcores=NS)
_sc_params = pltpu.CompilerParams(use_tc_tiling_on_sc=False)

_T0 = (((0,), (0,)), ((), ()))

GROUPS = [(g * 128, 128) for g in range(7)] + [(896, 104)]


def _stripe(cid):
    row0 = (cid // 16) * PB + (cid % 2) * CHUNK
    col0 = ((cid % 16) // 2) * 16
    return row0, col0


def _node_mlp_body(xt_ref, bd_ref, bcat_ref, mW1_ref, mb1_ref, mW2_ref,
                   mb2_ref, h_ref):
    w0 = bd_ref[...] @ mW1_ref[...]
    b0 = bcat_ref[...] @ mW1_ref[...] + mb1_ref[...]
    z = lax.dot_general(xt_ref[...], w0, _T0,
                        preferred_element_type=jnp.float32)
    h1 = jnp.maximum(z + b0, 0.0)
    h_ref[...] = h1 @ mW2_ref[...] + mb2_ref[...]


def _node_mlp(xt, bd, bcat, mW1, mb1, mW2, mb2):
    return pl.pallas_call(
        _node_mlp_body,
        grid=(1,),
        in_specs=[
            pl.BlockSpec((51, N), lambda i: (0, 0)),
            pl.BlockSpec((51, 24), lambda i: (0, 0)),
            pl.BlockSpec((1, 24), lambda i: (0, 0)),
            pl.BlockSpec((24, H), lambda i: (0, 0)),
            pl.BlockSpec((1, H), lambda i: (0, 0)),
            pl.BlockSpec((H, H), lambda i: (0, 0)),
            pl.BlockSpec((1, H), lambda i: (0, 0)),
        ],
        out_specs=pl.BlockSpec((N, H), lambda i: (0, 0)),
        out_shape=jax.ShapeDtypeStruct((N, H), jnp.float32),
    )(xt, bd, bcat, mW1, mb1, mW2, mb2)


@functools.partial(
    pl.kernel,
    out_type=jax.ShapeDtypeStruct((EP8, 128), jnp.float32),
    mesh=_mesh,
    compiler_params=_sc_params,
    scratch_types=[
        pltpu.VMEM((2, CHUNK), jnp.int32),
        pltpu.VMEM((2, CHUNK), jnp.int32),
        pltpu.VMEM((2, CHUNK, 16), jnp.float32),
        pltpu.VMEM((2, CHUNK, 16), jnp.float32),
        pltpu.SemaphoreType.DMA,
        pltpu.SemaphoreType.DMA,
        pltpu.SemaphoreType.DMA,
        pltpu.SemaphoreType.DMA,
        pltpu.SemaphoreType.DMA,
        pltpu.SemaphoreType.DMA,
    ],
)
def _dist_kernel(posp_hbm, ei_hbm, sq_hbm, sidx, didx, ps, pd,
                 isem0, isem1, gsem0, gsem1, wsem0, wsem1):
    c = lax.axis_index("c")
    s = lax.axis_index("s")
    wid = c * NS + s
    isem = (isem0, isem1)
    gsem = (gsem0, gsem1)
    wsem = (wsem0, wsem1)

    def fire_idx(k):
        sl = k % 2
        base = (wid * NCHUNK + k) * CHUNK
        return [
            pltpu.async_copy(ei_hbm.at[0, pl.ds(base, CHUNK)], sidx.at[sl],
                             isem[sl]),
            pltpu.async_copy(ei_hbm.at[1, pl.ds(base, CHUNK)], didx.at[sl],
                             isem[sl]),
        ]

    def fire_gather(k):
        sl = k % 2
        ds_ = []
        for off, cnt in GROUPS:
            ds_.append(pltpu.async_copy(
                posp_hbm.at[sidx.at[sl, pl.ds(off, cnt)]],
                ps.at[sl, pl.ds(off, cnt)], gsem[sl]))
            ds_.append(pltpu.async_copy(
                posp_hbm.at[didx.at[sl, pl.ds(off, cnt)]],
                pd.at[sl, pl.ds(off, cnt)], gsem[sl]))
        return ds_

    def fire_write(k):
        sl = k % 2
        cid = wid * NCHUNK + k
        row0, col0 = _stripe(cid)
        return [pltpu.async_copy(
            ps.at[sl],
            sq_hbm.at[pl.ds(row0, CHUNK), pl.ds(col0, 16)],
            wsem[sl])]

    idx_d = {0: fire_idx(0), 1: fire_idx(1)}
    for d in idx_d[0]:
        d.wait()
    gat_d = {0: fire_gather(0)}
    wr_d = {}
    for k in range(NCHUNK):
        sl = k % 2
        for d in gat_d[k]:
            d.wait()
        if k + 2 < NCHUNK:
            idx_d[k + 2] = fire_idx(k + 2)
        if k - 1 in wr_d:
            for d in wr_d[k - 1]:
                d.wait()
        if k + 1 < NCHUNK:
            for d in idx_d[k + 1]:
                d.wait()
            gat_d[k + 1] = fire_gather(k + 1)

        def sqd(i, carry2):
            v = ps[sl, i] - pd[sl, i]
            ps[sl, i] = v * v
            return carry2

        lax.fori_loop(0, CHUNK, sqd, 0)
        wr_d[k] = fire_write(k)
    for d in wr_d[NCHUNK - 1]:
        d.wait()


def _edge_mlp_body(eat_ref, sqp_ref, eW1a_ref, w1bd_ref, b1_ref, w2bd_ref,
                   b2_ref, e_ref):
    delta = 5.0 / (DISTE - 1)
    coeff = -0.5 / (delta * delta)
    sqp = sqp_ref[...]
    S = (lax.broadcasted_iota(jnp.int32, (128, 8), 0) // 16 ==
         lax.broadcasted_iota(jnp.int32, (128, 8), 1)).astype(jnp.float32)
    d2 = sqp @ S
    d = jnp.sqrt(d2 + 1e-12)
    R = (lax.broadcasted_iota(jnp.int32, (8, 256), 0) ==
         lax.broadcasted_iota(jnp.int32, (8, 256), 1) // DISTE
         ).astype(jnp.float32)
    dbc = d @ R
    offs = (lax.broadcasted_iota(jnp.int32, (1, 256), 1) % DISTE
            ).astype(jnp.float32) * delta
    dist = jnp.exp(coeff * (dbc - offs) ** 2)
    contrib = dist @ w1bd_ref[...]
    eat = eat_ref[...]
    w1a = eW1a_ref[...]
    zs = [lax.dot_general(eat[:, PB * g:PB * (g + 1)], w1a, _T0,
                          preferred_element_type=jnp.float32)
          for g in range(8)]
    z_ea = jnp.concatenate(zs, axis=1)
    z = jnp.maximum(z_ea + contrib + b1_ref[...], 0.0)
    e_ref[...] = z @ w2bd_ref[...] + b2_ref[...]


def _edge_mlp(eat, sq_p, eW1a, w1bd, b1rep, w2bd, b2rep):
    grid = E // TCB
    return pl.pallas_call(
        _edge_mlp_body,
        grid=(grid,),
        in_specs=[
            pl.BlockSpec((LIG_E + SIGMA, TCB), lambda i: (0, i)),
            pl.BlockSpec((PB, 128), lambda i: (i, 0)),
            pl.BlockSpec((LIG_E + SIGMA, H), lambda i: (0, 0)),
            pl.BlockSpec((8 * DISTE, 128), lambda i: (0, 0)),
            pl.BlockSpec((1, 128), lambda i: (0, 0)),
            pl.BlockSpec((128, 128), lambda i: (0, 0)),
            pl.BlockSpec((1, 128), lambda i: (0, 0)),
        ],
        out_specs=pl.BlockSpec((PB, 128), lambda i: (i, 0)),
        out_shape=jax.ShapeDtypeStruct((EP8, 128), jnp.float32),
    )(eat, sq_p, eW1a, w1bd, b1rep, w2bd, b2rep)


@functools.partial(
    pl.kernel,
    out_type=jax.ShapeDtypeStruct((NC, NPAD, H), jnp.float32),
    mesh=_mesh,
    compiler_params=_sc_params,
    scratch_types=[
        pltpu.VMEM((2, CHUNK), jnp.int32),
        pltpu.VMEM((2, NG, G), jnp.int32),
        pltpu.VMEM((2, CHUNK, H), jnp.float32),
        pltpu.VMEM((2, CHUNK, H), jnp.float32),
        pltpu.VMEM_SHARED((NPAD, H), jnp.float32),
        pltpu.SemaphoreType.DMA,
        pltpu.SemaphoreType.DMA,
        pltpu.SemaphoreType.DMA,
        pltpu.SemaphoreType.DMA,
        pltpu.SemaphoreType.DMA,
        pltpu.SemaphoreType.DMA,
    ],
)
def _scatter_kernel(h_hbm, e_hbm, ei_hbm, dst_hbm, zeros_hbm, out_hbm,
                    sidx, didx, eb, hb, accum,
                    isem0, isem1, gsem0, gsem1, ssem0, ssem1):
    c = lax.axis_index("c")
    s = lax.axis_index("s")
    wid = c * NS + s
    isem = (isem0, isem1)
    gsem = (gsem0, gsem1)
    ssem = (ssem0, ssem1)

    def fire_loads(k):
        sl = k % 2
        cid = wid * NCHUNK + k
        base = cid * CHUNK
        base_rows = cid * NG
        row0, col0 = _stripe(cid)
        return [
            pltpu.async_copy(ei_hbm.at[0, pl.ds(base, CHUNK)], sidx.at[sl],
                             isem[sl]),
            pltpu.async_copy(dst_hbm.at[pl.ds(base_rows, NG)], didx.at[sl],
                             isem[sl]),
            pltpu.async_copy(
                e_hbm.at[pl.ds(row0, CHUNK), pl.ds(col0, 16)],
                eb.at[sl], isem[sl]),
        ]

    def fire_gather(k):
        sl = k % 2
        return [
            pltpu.async_copy(h_hbm.at[sidx.at[sl, pl.ds(off, cnt)]],
                             hb.at[sl, pl.ds(off, cnt)], gsem[sl])
            for off, cnt in GROUPS
        ]

    def fire_scatter(k):
        sl = k % 2
        return [
            pltpu.async_copy(hb.at[sl, pl.ds(g * G, G)],
                             accum.at[didx.at[sl, g]], ssem[sl], add=True)
            for g in range(NG)
        ]

    ld_d = {0: fire_loads(0)}
    pltpu.sync_copy(zeros_hbm, accum.at[pl.ds(s * RPT, RPT)])
    plsc.subcore_barrier()
    for d in ld_d[0]:
        d.wait()
    gat_d = {0: fire_gather(0)}
    sc_d = {}
    for k in range(NCHUNK):
        sl = k % 2
        for d in gat_d[k]:
            d.wait()
        if k - 1 in sc_d:
            for d in sc_d[k - 1]:
                d.wait()
        if k + 1 < NCHUNK:
            ld_d[k + 1] = fire_loads(k + 1)

        def mul(i, carry2):
            hb[sl, i] = hb[sl, i] * eb[sl, i]
            return carry2

        lax.fori_loop(0, CHUNK, mul, 0)
        if k + 1 < NCHUNK:
            for d in ld_d[k + 1]:
                d.wait()
            gat_d[k + 1] = fire_gather(k + 1)
        sc_d[k] = fire_scatter(k)
    for d in sc_d[NCHUNK - 1]:
        d.wait()
    plsc.subcore_barrier()
    pltpu.sync_copy(accum.at[pl.ds(s * RPT, RPT)],
                    out_hbm.at[c, pl.ds(s * RPT, RPT)])


NROW = NPAD // NW
NLAST = N - (NW - 1) * NROW


@functools.partial(
    pl.kernel,
    out_type=jax.ShapeDtypeStruct((N, H), jnp.float32),
    mesh=_mesh,
    compiler_params=_sc_params,
    scratch_types=[
        pltpu.VMEM((NROW, H), jnp.float32),
        pltpu.VMEM((NROW, H), jnp.float32),
        pltpu.VMEM((NROW, H), jnp.float32),
        pltpu.SemaphoreType.DMA,
    ],
)
def _final_kernel(h_hbm, agg_hbm, o_hbm, hb, a0, a1, sem):
    c = lax.axis_index("c")
    s = lax.axis_index("s")
    wid = c * NS + s
    r0 = wid * NROW

    def doit(nrow):
        ds_ = [
            pltpu.async_copy(h_hbm.at[pl.ds(r0, nrow)],
                             hb.at[pl.ds(0, nrow)], sem),
            pltpu.async_copy(agg_hbm.at[0, pl.ds(r0, nrow)],
                             a0.at[pl.ds(0, nrow)], sem),
            pltpu.async_copy(agg_hbm.at[1, pl.ds(r0, nrow)],
                             a1.at[pl.ds(0, nrow)], sem),
        ]
        for d in ds_:
            d.wait()

        def add(i, cy):
            hb[i] = hb[i] + (a0[i] + a1[i])
            return cy

        lax.fori_loop(0, nrow, add, 0)
        pltpu.sync_copy(hb.at[pl.ds(0, nrow)], o_hbm.at[pl.ds(r0, nrow)])

    @pl.when(wid < NW - 1)
    def _():
        doit(NROW)

    @pl.when(wid == NW - 1)
    def _():
        doit(NLAST)


def kernel(x, pos, edge_attr, edge_index, pW, pb, fW, fb, tW, tb, mW1, mb1,
           mW2, mb2, eW1, eb1, eW2, eb2):
    posp = jnp.pad(pos, ((0, 0), (0, 16 - 3)))
    xt = jnp.swapaxes(x, 0, 1)
    eat = jnp.swapaxes(edge_attr, 0, 1)
    dst2d = edge_index[1].reshape(E // G, G)
    bd = jax.scipy.linalg.block_diag(pW, fW, tW)
    bcat = jnp.concatenate([pb, fb, tb]).reshape(1, 24)
    eW1a = eW1[:LIG_E + SIGMA]
    w1bd = jax.scipy.linalg.block_diag(*([eW1[LIG_E + SIGMA:]] * 8))
    w2bd = jax.scipy.linalg.block_diag(*([eW2] * 8)) * (1.0 / 1000.0)
    b1rep = jnp.tile(eb1, 8).reshape(1, 128)
    b2rep = jnp.tile(eb2, 8).reshape(1, 128) * (1.0 / 1000.0)
    zeros_rpt = jnp.zeros((RPT, H), jnp.float32)

    h = _node_mlp(xt, bd, bcat, mW1, mb1.reshape(1, H), mW2,
                  mb2.reshape(1, H))
    sq_p = _dist_kernel(posp, edge_index)
    e_p = _edge_mlp(eat, sq_p, eW1a, w1bd, b1rep, w2bd, b2rep)
    agg = _scatter_kernel(h, e_p, edge_index, dst2d, zeros_rpt)
    return _final_kernel(h, agg)


# speedup vs baseline: 10.3010x; 1.0020x over previous
import functools

import jax
import jax.numpy as jnp
from jax import lax
from jax.experimental import pallas as pl
from jax.experimental.pallas import tpu as pltpu
from jax.experimental.pallas import tpu_sc as plsc

N = 50000
E = 800000
NCAT = 16
SIGMA = 32
DISTE = 32
LIG_E = 4
H = 16

NC = 2
NS = 16
NW = NC * NS
CHUNK = 1000
NCHUNK = E // CHUNK // NW
G = 125
NG = CHUNK // G
EP8 = E // 8
TCB = 32000
PB = TCB // 8
SPB = PB // CHUNK
NPAD = 50048
RPT = NPAD // NS

_mesh = plsc.VectorSubcoreMesh(
    core_axis_name="c", subcore_axis_name="s", num_cores=NC, num_subcores=NS)
_sc_params = pltpu.CompilerParams(use_tc_tiling_on_sc=False)

_T0 = (((0,), (0,)), ((), ()))

GROUPS = [(g * 128, 128) for g in range(7)] + [(896, 104)]


def _stripe(cid):
    row0 = (cid // (8 * SPB)) * PB + (cid % SPB) * CHUNK
    col0 = ((cid % (8 * SPB)) // SPB) * 16
    return row0, col0


def _node_mlp_body(xt_ref, bd_ref, bcat_ref, mW1_ref, mb1_ref, mW2_ref,
                   mb2_ref, h_ref):
    w0 = bd_ref[...] @ mW1_ref[...]
    b0 = bcat_ref[...] @ mW1_ref[...] + mb1_ref[...]
    z = lax.dot_general(xt_ref[...], w0, _T0,
                        preferred_element_type=jnp.float32)
    h1 = jnp.maximum(z + b0, 0.0)
    h_ref[...] = h1 @ mW2_ref[...] + mb2_ref[...]


def _node_mlp(xt, bd, bcat, mW1, mb1, mW2, mb2):
    return pl.pallas_call(
        _node_mlp_body,
        grid=(1,),
        in_specs=[
            pl.BlockSpec((51, N), lambda i: (0, 0)),
            pl.BlockSpec((51, 24), lambda i: (0, 0)),
            pl.BlockSpec((1, 24), lambda i: (0, 0)),
            pl.BlockSpec((24, H), lambda i: (0, 0)),
            pl.BlockSpec((1, H), lambda i: (0, 0)),
            pl.BlockSpec((H, H), lambda i: (0, 0)),
            pl.BlockSpec((1, H), lambda i: (0, 0)),
        ],
        out_specs=pl.BlockSpec((N, H), lambda i: (0, 0)),
        out_shape=jax.ShapeDtypeStruct((N, H), jnp.float32),
    )(xt, bd, bcat, mW1, mb1, mW2, mb2)


@functools.partial(
    pl.kernel,
    out_type=jax.ShapeDtypeStruct((EP8, 128), jnp.float32),
    mesh=_mesh,
    compiler_params=_sc_params,
    scratch_types=[
        pltpu.VMEM((3, CHUNK), jnp.int32),
        pltpu.VMEM((3, CHUNK), jnp.int32),
        pltpu.VMEM((3, CHUNK, 16), jnp.float32),
        pltpu.VMEM((3, CHUNK, 16), jnp.float32),
        pltpu.SemaphoreType.DMA,
        pltpu.SemaphoreType.DMA,
        pltpu.SemaphoreType.DMA,
        pltpu.SemaphoreType.DMA,
        pltpu.SemaphoreType.DMA,
        pltpu.SemaphoreType.DMA,
        pltpu.SemaphoreType.DMA,
        pltpu.SemaphoreType.DMA,
        pltpu.SemaphoreType.DMA,
    ],
)
def _dist_kernel(posp_hbm, ei_hbm, sq_hbm, sidx, didx, ps, pd,
                 isem0, isem1, isem2, gsem0, gsem1, gsem2,
                 wsem0, wsem1, wsem2):
    c = lax.axis_index("c")
    s = lax.axis_index("s")
    wid = c * NS + s
    isem = (isem0, isem1, isem2)
    gsem = (gsem0, gsem1, gsem2)
    wsem = (wsem0, wsem1, wsem2)

    def fire_idx(k):
        sl = k % 3
        base = (wid * NCHUNK + k) * CHUNK
        return [
            pltpu.async_copy(ei_hbm.at[0, pl.ds(base, CHUNK)], sidx.at[sl],
                             isem[sl]),
            pltpu.async_copy(ei_hbm.at[1, pl.ds(base, CHUNK)], didx.at[sl],
                             isem[sl]),
        ]

    def fire_gather(k):
        sl = k % 3
        ds_ = []
        for off, cnt in GROUPS:
            ds_.append(pltpu.async_copy(
                posp_hbm.at[sidx.at[sl, pl.ds(off, cnt)]],
                ps.at[sl, pl.ds(off, cnt)], gsem[sl]))
            ds_.append(pltpu.async_copy(
                posp_hbm.at[didx.at[sl, pl.ds(off, cnt)]],
                pd.at[sl, pl.ds(off, cnt)], gsem[sl]))
        return ds_

    def fire_write(k):
        sl = k % 3
        cid = wid * NCHUNK + k
        row0, col0 = _stripe(cid)
        return [pltpu.async_copy(
            ps.at[sl],
            sq_hbm.at[pl.ds(row0, CHUNK), pl.ds(col0, 16)],
            wsem[sl])]

    idx_d = {0: fire_idx(0), 1: fire_idx(1), 2: fire_idx(2)}
    for d in idx_d[0]:
        d.wait()
    gat_d = {0: fire_gather(0)}
    for d in idx_d[1]:
        d.wait()
    gat_d[1] = fire_gather(1)
    wr_d = {}
    for k in range(NCHUNK):
        sl = k % 3
        for d in gat_d[k]:
            d.wait()
        if k + 3 < NCHUNK:
            idx_d[k + 3] = fire_idx(k + 3)
        if k - 1 in wr_d:
            for d in wr_d[k - 1]:
                d.wait()
        if k + 2 < NCHUNK:
            for d in idx_d[k + 2]:
                d.wait()
            gat_d[k + 2] = fire_gather(k + 2)

        def sqd(i, carry2):
            v = ps[sl, i] - pd[sl, i]
            ps[sl, i] = v * v
            return carry2

        lax.fori_loop(0, CHUNK, sqd, 0)
        wr_d[k] = fire_write(k)
    for d in wr_d[NCHUNK - 1]:
        d.wait()


def _edge_mlp_body(eat_ref, sqp_ref, eW1a_ref, w1bd_ref, b1_ref, w2bd_ref,
                   b2_ref, e_ref):
    delta = 5.0 / (DISTE - 1)
    coeff = -0.5 / (delta * delta)
    sqp = sqp_ref[...]
    S = (lax.broadcasted_iota(jnp.int32, (128, 8), 0) // 16 ==
         lax.broadcasted_iota(jnp.int32, (128, 8), 1)).astype(jnp.float32)
    d2 = sqp @ S
    d = jnp.sqrt(d2 + 1e-12)
    R = (lax.broadcasted_iota(jnp.int32, (8, 256), 0) ==
         lax.broadcasted_iota(jnp.int32, (8, 256), 1) // DISTE
         ).astype(jnp.float32)
    dbc = d @ R
    offs = (lax.broadcasted_iota(jnp.int32, (1, 256), 1) % DISTE
            ).astype(jnp.float32) * delta
    dist = jnp.exp(coeff * (dbc - offs) ** 2)
    contrib = dist @ w1bd_ref[...]
    eat = eat_ref[...]
    w1a = eW1a_ref[...]
    zs = [lax.dot_general(eat[:, PB * g:PB * (g + 1)], w1a, _T0,
                          preferred_element_type=jnp.float32)
          for g in range(8)]
    z_ea = jnp.concatenate(zs, axis=1)
    z = jnp.maximum(z_ea + contrib + b1_ref[...], 0.0)
    e_ref[...] = z @ w2bd_ref[...] + b2_ref[...]


def _edge_mlp(eat, sq_p, eW1a, w1bd, b1rep, w2bd, b2rep):
    grid = E // TCB
    return pl.pallas_call(
        _edge_mlp_body,
        grid=(grid,),
        in_specs=[
            pl.BlockSpec((LIG_E + SIGMA, TCB), lambda i: (0, i)),
            pl.BlockSpec((PB, 128), lambda i: (i, 0)),
            pl.BlockSpec((LIG_E + SIGMA, H), lambda i: (0, 0)),
            pl.BlockSpec((8 * DISTE, 128), lambda i: (0, 0)),
            pl.BlockSpec((1, 128), lambda i: (0, 0)),
            pl.BlockSpec((128, 128), lambda i: (0, 0)),
            pl.BlockSpec((1, 128), lambda i: (0, 0)),
        ],
        out_specs=pl.BlockSpec((PB, 128), lambda i: (i, 0)),
        out_shape=jax.ShapeDtypeStruct((EP8, 128), jnp.float32),
    )(eat, sq_p, eW1a, w1bd, b1rep, w2bd, b2rep)


@functools.partial(
    pl.kernel,
    out_type=jax.ShapeDtypeStruct((NC, NPAD, H), jnp.float32),
    mesh=_mesh,
    compiler_params=_sc_params,
    scratch_types=[
        pltpu.VMEM((2, CHUNK), jnp.int32),
        pltpu.VMEM((2, NG, G), jnp.int32),
        pltpu.VMEM((2, CHUNK, H), jnp.float32),
        pltpu.VMEM((2, CHUNK, H), jnp.float32),
        pltpu.VMEM_SHARED((NPAD, H), jnp.float32),
        pltpu.SemaphoreType.DMA,
        pltpu.SemaphoreType.DMA,
        pltpu.SemaphoreType.DMA,
        pltpu.SemaphoreType.DMA,
        pltpu.SemaphoreType.DMA,
        pltpu.SemaphoreType.DMA,
    ],
)
def _scatter_kernel(h_hbm, e_hbm, ei_hbm, dst_hbm, zeros_hbm, out_hbm,
                    sidx, didx, eb, hb, accum,
                    isem0, isem1, gsem0, gsem1, ssem0, ssem1):
    c = lax.axis_index("c")
    s = lax.axis_index("s")
    wid = c * NS + s
    isem = (isem0, isem1)
    gsem = (gsem0, gsem1)
    ssem = (ssem0, ssem1)

    def fire_loads(k):
        sl = k % 2
        cid = wid * NCHUNK + k
        base = cid * CHUNK
        base_rows = cid * NG
        row0, col0 = _stripe(cid)
        return [
            pltpu.async_copy(ei_hbm.at[0, pl.ds(base, CHUNK)], sidx.at[sl],
                             isem[sl]),
            pltpu.async_copy(dst_hbm.at[pl.ds(base_rows, NG)], didx.at[sl],
                             isem[sl]),
            pltpu.async_copy(
                e_hbm.at[pl.ds(row0, CHUNK), pl.ds(col0, 16)],
                eb.at[sl], isem[sl]),
        ]

    def fire_gather(k):
        sl = k % 2
        return [
            pltpu.async_copy(h_hbm.at[sidx.at[sl, pl.ds(off, cnt)]],
                             hb.at[sl, pl.ds(off, cnt)], gsem[sl])
            for off, cnt in GROUPS
        ]

    def fire_scatter(k):
        sl = k % 2
        return [
            pltpu.async_copy(hb.at[sl, pl.ds(g * G, G)],
                             accum.at[didx.at[sl, g]], ssem[sl], add=True)
            for g in range(NG)
        ]

    ld_d = {0: fire_loads(0)}
    pltpu.sync_copy(zeros_hbm, accum.at[pl.ds(s * RPT, RPT)])
    plsc.subcore_barrier()
    for d in ld_d[0]:
        d.wait()
    gat_d = {0: fire_gather(0)}
    sc_d = {}
    for k in range(NCHUNK):
        sl = k % 2
        for d in gat_d[k]:
            d.wait()
        if k - 1 in sc_d:
            for d in sc_d[k - 1]:
                d.wait()
        if k + 1 < NCHUNK:
            ld_d[k + 1] = fire_loads(k + 1)

        def mul(i, carry2):
            hb[sl, i] = hb[sl, i] * eb[sl, i]
            return carry2

        lax.fori_loop(0, CHUNK, mul, 0)
        if k + 1 < NCHUNK:
            for d in ld_d[k + 1]:
                d.wait()
            gat_d[k + 1] = fire_gather(k + 1)
        sc_d[k] = fire_scatter(k)
    for d in sc_d[NCHUNK - 1]:
        d.wait()
    plsc.subcore_barrier()
    pltpu.sync_copy(accum.at[pl.ds(s * RPT, RPT)],
                    out_hbm.at[c, pl.ds(s * RPT, RPT)])


NROW = NPAD // NW
NLAST = N - (NW - 1) * NROW


@functools.partial(
    pl.kernel,
    out_type=jax.ShapeDtypeStruct((N, H), jnp.float32),
    mesh=_mesh,
    compiler_params=_sc_params,
    scratch_types=[
        pltpu.VMEM((NROW, H), jnp.float32),
        pltpu.VMEM((NROW, H), jnp.float32),
        pltpu.VMEM((NROW, H), jnp.float32),
        pltpu.SemaphoreType.DMA,
    ],
)
def _final_kernel(h_hbm, agg_hbm, o_hbm, hb, a0, a1, sem):
    c = lax.axis_index("c")
    s = lax.axis_index("s")
    wid = c * NS + s
    r0 = wid * NROW

    def doit(nrow):
        ds_ = [
            pltpu.async_copy(h_hbm.at[pl.ds(r0, nrow)],
                             hb.at[pl.ds(0, nrow)], sem),
            pltpu.async_copy(agg_hbm.at[0, pl.ds(r0, nrow)],
                             a0.at[pl.ds(0, nrow)], sem),
            pltpu.async_copy(agg_hbm.at[1, pl.ds(r0, nrow)],
                             a1.at[pl.ds(0, nrow)], sem),
        ]
        for d in ds_:
            d.wait()

        def add(i, cy):
            hb[i] = hb[i] + (a0[i] + a1[i])
            return cy

        lax.fori_loop(0, nrow, add, 0)
        pltpu.sync_copy(hb.at[pl.ds(0, nrow)], o_hbm.at[pl.ds(r0, nrow)])

    @pl.when(wid < NW - 1)
    def _():
        doit(NROW)

    @pl.when(wid == NW - 1)
    def _():
        doit(NLAST)


def kernel(x, pos, edge_attr, edge_index, pW, pb, fW, fb, tW, tb, mW1, mb1,
           mW2, mb2, eW1, eb1, eW2, eb2):
    posp = jnp.pad(pos, ((0, 0), (0, 16 - 3)))
    xt = jnp.swapaxes(x, 0, 1)
    eat = jnp.swapaxes(edge_attr, 0, 1)
    dst2d = edge_index[1].reshape(E // G, G)
    bd = jax.scipy.linalg.block_diag(pW, fW, tW)
    bcat = jnp.concatenate([pb, fb, tb]).reshape(1, 24)
    eW1a = eW1[:LIG_E + SIGMA]
    w1bd = jax.scipy.linalg.block_diag(*([eW1[LIG_E + SIGMA:]] * 8))
    w2bd = jax.scipy.linalg.block_diag(*([eW2] * 8)) * (1.0 / 1000.0)
    b1rep = jnp.tile(eb1, 8).reshape(1, 128)
    b2rep = jnp.tile(eb2, 8).reshape(1, 128) * (1.0 / 1000.0)
    zeros_rpt = jnp.zeros((RPT, H), jnp.float32)

    h = _node_mlp(xt, bd, bcat, mW1, mb1.reshape(1, H), mW2,
                  mb2.reshape(1, H))
    sq_p = _dist_kernel(posp, edge_index)
    e_p = _edge_mlp(eat, sq_p, eW1a, w1bd, b1rep, w2bd, b2rep)
    agg = _scatter_kernel(h, e_p, edge_index, dst2d, zeros_rpt)
    return _final_kernel(h, agg)


# speedup vs baseline: 13.0494x; 1.2668x over previous
import functools

import jax
import jax.numpy as jnp
from jax import lax
from jax.experimental import pallas as pl
from jax.experimental.pallas import tpu as pltpu
from jax.experimental.pallas import tpu_sc as plsc

N = 50000
E = 800000
NCAT = 16
SIGMA = 32
DISTE = 32
LIG_E = 4
H = 16

NC = 2
NS = 16
NW = NC * NS
CHUNK = 1000
NCHUNK = E // CHUNK // NW
G = 125
NG = CHUNK // G
EP8 = E // 8
TCB = 32000
PB = TCB // 8
SPB = PB // CHUNK
NPAD = 50048
RPT = NPAD // NS

_mesh = plsc.VectorSubcoreMesh(
    core_axis_name="c", subcore_axis_name="s", num_cores=NC, num_subcores=NS)
_sc_params = pltpu.CompilerParams(use_tc_tiling_on_sc=False)

_T0 = (((0,), (0,)), ((), ()))

GROUPS = [(g * 128, 128) for g in range(7)] + [(896, 104)]


def _stripe(cid):
    row0 = (cid // (8 * SPB)) * PB + (cid % SPB) * CHUNK
    col0 = ((cid % (8 * SPB)) // SPB) * 16
    return row0, col0


def _node_mlp_body(xt_ref, bd_ref, bcat_ref, mW1_ref, mb1_ref, mW2_ref,
                   mb2_ref, h_ref):
    w0 = bd_ref[...] @ mW1_ref[...]
    b0 = bcat_ref[...] @ mW1_ref[...] + mb1_ref[...]
    z = lax.dot_general(xt_ref[...], w0, _T0,
                        preferred_element_type=jnp.float32)
    h1 = jnp.maximum(z + b0, 0.0)
    h_ref[...] = h1 @ mW2_ref[...] + mb2_ref[...]


def _node_mlp(xt, bd, bcat, mW1, mb1, mW2, mb2):
    return pl.pallas_call(
        _node_mlp_body,
        grid=(1,),
        in_specs=[
            pl.BlockSpec((51, N), lambda i: (0, 0)),
            pl.BlockSpec((51, 24), lambda i: (0, 0)),
            pl.BlockSpec((1, 24), lambda i: (0, 0)),
            pl.BlockSpec((24, H), lambda i: (0, 0)),
            pl.BlockSpec((1, H), lambda i: (0, 0)),
            pl.BlockSpec((H, H), lambda i: (0, 0)),
            pl.BlockSpec((1, H), lambda i: (0, 0)),
        ],
        out_specs=pl.BlockSpec((N, H), lambda i: (0, 0)),
        out_shape=jax.ShapeDtypeStruct((N, H), jnp.float32),
    )(xt, bd, bcat, mW1, mb1, mW2, mb2)


def _make_dist(nchunk, cid_base, row_base, out_rows):
  @functools.partial(
    pl.kernel,
    out_type=jax.ShapeDtypeStruct((out_rows, 128), jnp.float32),
    mesh=_mesh,
    compiler_params=_sc_params,
    scratch_types=[
        pltpu.VMEM((3, CHUNK), jnp.int32),
        pltpu.VMEM((3, CHUNK), jnp.int32),
        pltpu.VMEM((3, CHUNK, 16), jnp.float32),
        pltpu.VMEM((3, CHUNK, 16), jnp.float32),
        pltpu.SemaphoreType.DMA,
        pltpu.SemaphoreType.DMA,
        pltpu.SemaphoreType.DMA,
        pltpu.SemaphoreType.DMA,
        pltpu.SemaphoreType.DMA,
        pltpu.SemaphoreType.DMA,
        pltpu.SemaphoreType.DMA,
        pltpu.SemaphoreType.DMA,
        pltpu.SemaphoreType.DMA,
    ],
  )
  def _dist_kernel(posp_hbm, ei_hbm, sq_hbm, sidx, didx, ps, pd,
                 isem0, isem1, isem2, gsem0, gsem1, gsem2,
                 wsem0, wsem1, wsem2):
    c = lax.axis_index("c")
    s = lax.axis_index("s")
    wid = c * NS + s
    isem = (isem0, isem1, isem2)
    gsem = (gsem0, gsem1, gsem2)
    wsem = (wsem0, wsem1, wsem2)

    def fire_idx(k):
        sl = k % 3
        base = (cid_base + wid * nchunk + k) * CHUNK
        return [
            pltpu.async_copy(ei_hbm.at[0, pl.ds(base, CHUNK)], sidx.at[sl],
                             isem[sl]),
            pltpu.async_copy(ei_hbm.at[1, pl.ds(base, CHUNK)], didx.at[sl],
                             isem[sl]),
        ]

    def fire_gather(k):
        sl = k % 3
        ds_ = []
        for off, cnt in GROUPS:
            ds_.append(pltpu.async_copy(
                posp_hbm.at[sidx.at[sl, pl.ds(off, cnt)]],
                ps.at[sl, pl.ds(off, cnt)], gsem[sl]))
            ds_.append(pltpu.async_copy(
                posp_hbm.at[didx.at[sl, pl.ds(off, cnt)]],
                pd.at[sl, pl.ds(off, cnt)], gsem[sl]))
        return ds_

    def fire_write(k):
        sl = k % 3
        cid = cid_base + wid * nchunk + k
        row0, col0 = _stripe(cid)
        row0 = row0 - row_base
        return [pltpu.async_copy(
            ps.at[sl],
            sq_hbm.at[pl.ds(row0, CHUNK), pl.ds(col0, 16)],
            wsem[sl])]

    idx_d = {0: fire_idx(0), 1: fire_idx(1), 2: fire_idx(2)}
    for d in idx_d[0]:
        d.wait()
    gat_d = {0: fire_gather(0)}
    for d in idx_d[1]:
        d.wait()
    gat_d[1] = fire_gather(1)
    wr_d = {}
    for k in range(nchunk):
        sl = k % 3
        for d in gat_d[k]:
            d.wait()
        if k + 3 < nchunk:
            idx_d[k + 3] = fire_idx(k + 3)
        if k - 1 in wr_d:
            for d in wr_d[k - 1]:
                d.wait()
        if k + 2 < nchunk:
            for d in idx_d[k + 2]:
                d.wait()
            gat_d[k + 2] = fire_gather(k + 2)

        def sqd(i, carry2):
            v = ps[sl, i] - pd[sl, i]
            ps[sl, i] = v * v
            return carry2

        lax.fori_loop(0, CHUNK, sqd, 0)
        wr_d[k] = fire_write(k)
    for d in wr_d[nchunk - 1]:
        d.wait()

  return _dist_kernel


_dist1 = _make_dist(12, 0, 0, 48000)
_dist2 = _make_dist(13, 384, 48000, 52000)


def _edge_mlp_body(eat_ref, sqp_ref, eW1a_ref, w1bd_ref, b1_ref, w2bd_ref,
                   b2_ref, e_ref):
    delta = 5.0 / (DISTE - 1)
    coeff = -0.5 / (delta * delta)
    sqp = sqp_ref[...]
    S = (lax.broadcasted_iota(jnp.int32, (128, 8), 0) // 16 ==
         lax.broadcasted_iota(jnp.int32, (128, 8), 1)).astype(jnp.float32)
    d2 = sqp @ S
    d = jnp.sqrt(d2 + 1e-12)
    R = (lax.broadcasted_iota(jnp.int32, (8, 256), 0) ==
         lax.broadcasted_iota(jnp.int32, (8, 256), 1) // DISTE
         ).astype(jnp.float32)
    dbc = d @ R
    offs = (lax.broadcasted_iota(jnp.int32, (1, 256), 1) % DISTE
            ).astype(jnp.float32) * delta
    dist = jnp.exp(coeff * (dbc - offs) ** 2)
    contrib = dist @ w1bd_ref[...]
    eat = eat_ref[...]
    w1a = eW1a_ref[...]
    zs = [lax.dot_general(eat[:, PB * g:PB * (g + 1)], w1a, _T0,
                          preferred_element_type=jnp.float32)
          for g in range(8)]
    z_ea = jnp.concatenate(zs, axis=1)
    z = jnp.maximum(z_ea + contrib + b1_ref[...], 0.0)
    e_ref[...] = z @ w2bd_ref[...] + b2_ref[...]


def _edge_mlp(eat, sq_p, eW1a, w1bd, b1rep, w2bd, b2rep, grid, blk0):
    return pl.pallas_call(
        _edge_mlp_body,
        grid=(grid,),
        in_specs=[
            pl.BlockSpec((LIG_E + SIGMA, TCB), lambda i, b=blk0: (0, i + b)),
            pl.BlockSpec((PB, 128), lambda i: (i, 0)),
            pl.BlockSpec((LIG_E + SIGMA, H), lambda i: (0, 0)),
            pl.BlockSpec((8 * DISTE, 128), lambda i: (0, 0)),
            pl.BlockSpec((1, 128), lambda i: (0, 0)),
            pl.BlockSpec((128, 128), lambda i: (0, 0)),
            pl.BlockSpec((1, 128), lambda i: (0, 0)),
        ],
        out_specs=pl.BlockSpec((PB, 128), lambda i: (i, 0)),
        out_shape=jax.ShapeDtypeStruct((grid * PB, 128), jnp.float32),
    )(eat, sq_p, eW1a, w1bd, b1rep, w2bd, b2rep)


def _make_scatter(nchunk, cid_base, row_base):
  @functools.partial(
    pl.kernel,
    out_type=jax.ShapeDtypeStruct((NC, NPAD, H), jnp.float32),
    mesh=_mesh,
    compiler_params=_sc_params,
    scratch_types=[
        pltpu.VMEM((2, CHUNK), jnp.int32),
        pltpu.VMEM((2, NG, G), jnp.int32),
        pltpu.VMEM((2, CHUNK, H), jnp.float32),
        pltpu.VMEM((2, CHUNK, H), jnp.float32),
        pltpu.VMEM_SHARED((NPAD, H), jnp.float32),
        pltpu.SemaphoreType.DMA,
        pltpu.SemaphoreType.DMA,
        pltpu.SemaphoreType.DMA,
        pltpu.SemaphoreType.DMA,
        pltpu.SemaphoreType.DMA,
        pltpu.SemaphoreType.DMA,
    ],
  )
  def _scatter_kernel(h_hbm, e_hbm, ei_hbm, dst_hbm, zeros_hbm, out_hbm,
                    sidx, didx, eb, hb, accum,
                    isem0, isem1, gsem0, gsem1, ssem0, ssem1):
    c = lax.axis_index("c")
    s = lax.axis_index("s")
    wid = c * NS + s
    isem = (isem0, isem1)
    gsem = (gsem0, gsem1)
    ssem = (ssem0, ssem1)

    def fire_loads(k):
        sl = k % 2
        cid = cid_base + wid * nchunk + k
        base = cid * CHUNK
        base_rows = cid * NG
        row0, col0 = _stripe(cid)
        row0 = row0 - row_base
        return [
            pltpu.async_copy(ei_hbm.at[0, pl.ds(base, CHUNK)], sidx.at[sl],
                             isem[sl]),
            pltpu.async_copy(dst_hbm.at[pl.ds(base_rows, NG)], didx.at[sl],
                             isem[sl]),
            pltpu.async_copy(
                e_hbm.at[pl.ds(row0, CHUNK), pl.ds(col0, 16)],
                eb.at[sl], isem[sl]),
        ]

    def fire_gather(k):
        sl = k % 2
        return [
            pltpu.async_copy(h_hbm.at[sidx.at[sl, pl.ds(off, cnt)]],
                             hb.at[sl, pl.ds(off, cnt)], gsem[sl])
            for off, cnt in GROUPS
        ]

    def fire_scatter(k):
        sl = k % 2
        return [
            pltpu.async_copy(hb.at[sl, pl.ds(g * G, G)],
                             accum.at[didx.at[sl, g]], ssem[sl], add=True)
            for g in range(NG)
        ]

    ld_d = {0: fire_loads(0)}
    pltpu.sync_copy(zeros_hbm, accum.at[pl.ds(s * RPT, RPT)])
    plsc.subcore_barrier()
    for d in ld_d[0]:
        d.wait()
    gat_d = {0: fire_gather(0)}
    sc_d = {}
    for k in range(nchunk):
        sl = k % 2
        for d in gat_d[k]:
            d.wait()
        if k - 1 in sc_d:
            for d in sc_d[k - 1]:
                d.wait()
        if k + 1 < nchunk:
            ld_d[k + 1] = fire_loads(k + 1)

        def mul(i, carry2):
            hb[sl, i] = hb[sl, i] * eb[sl, i]
            return carry2

        lax.fori_loop(0, CHUNK, mul, 0)
        if k + 1 < nchunk:
            for d in ld_d[k + 1]:
                d.wait()
            gat_d[k + 1] = fire_gather(k + 1)
        sc_d[k] = fire_scatter(k)
    for d in sc_d[nchunk - 1]:
        d.wait()
    plsc.subcore_barrier()
    pltpu.sync_copy(accum.at[pl.ds(s * RPT, RPT)],
                    out_hbm.at[c, pl.ds(s * RPT, RPT)])

  return _scatter_kernel


_scat1 = _make_scatter(12, 0, 0)
_scat2 = _make_scatter(13, 384, 48000)


NROW = NPAD // NW
NLAST = N - (NW - 1) * NROW


@functools.partial(
    pl.kernel,
    out_type=jax.ShapeDtypeStruct((N, H), jnp.float32),
    mesh=_mesh,
    compiler_params=_sc_params,
    scratch_types=[
        pltpu.VMEM((NROW, H), jnp.float32),
        pltpu.VMEM((NROW, H), jnp.float32),
        pltpu.VMEM((NROW, H), jnp.float32),
        pltpu.SemaphoreType.DMA,
    ],
)
def _final_kernel(h_hbm, agga_hbm, aggb_hbm, o_hbm, hb, a0, a1, sem):
    c = lax.axis_index("c")
    s = lax.axis_index("s")
    wid = c * NS + s
    r0 = wid * NROW

    def doit(nrow):
        ds_ = [
            pltpu.async_copy(h_hbm.at[pl.ds(r0, nrow)],
                             hb.at[pl.ds(0, nrow)], sem),
            pltpu.async_copy(agga_hbm.at[0, pl.ds(r0, nrow)],
                             a0.at[pl.ds(0, nrow)], sem),
            pltpu.async_copy(agga_hbm.at[1, pl.ds(r0, nrow)],
                             a1.at[pl.ds(0, nrow)], sem),
        ]
        for d in ds_:
            d.wait()

        def add(i, cy):
            hb[i] = hb[i] + (a0[i] + a1[i])
            return cy

        lax.fori_loop(0, nrow, add, 0)
        ds2_ = [
            pltpu.async_copy(aggb_hbm.at[0, pl.ds(r0, nrow)],
                             a0.at[pl.ds(0, nrow)], sem),
            pltpu.async_copy(aggb_hbm.at[1, pl.ds(r0, nrow)],
                             a1.at[pl.ds(0, nrow)], sem),
        ]
        for d in ds2_:
            d.wait()
        lax.fori_loop(0, nrow, add, 0)
        pltpu.sync_copy(hb.at[pl.ds(0, nrow)], o_hbm.at[pl.ds(r0, nrow)])

    @pl.when(wid < NW - 1)
    def _():
        doit(NROW)

    @pl.when(wid == NW - 1)
    def _():
        doit(NLAST)


def kernel(x, pos, edge_attr, edge_index, pW, pb, fW, fb, tW, tb, mW1, mb1,
           mW2, mb2, eW1, eb1, eW2, eb2):
    posp = jnp.pad(pos, ((0, 0), (0, 16 - 3)))
    xt = jnp.swapaxes(x, 0, 1)
    eat = jnp.swapaxes(edge_attr, 0, 1)
    dst2d = edge_index[1].reshape(E // G, G)
    bd = jax.scipy.linalg.block_diag(pW, fW, tW)
    bcat = jnp.concatenate([pb, fb, tb]).reshape(1, 24)
    eW1a = eW1[:LIG_E + SIGMA]
    w1bd = jax.scipy.linalg.block_diag(*([eW1[LIG_E + SIGMA:]] * 8))
    w2bd = jax.scipy.linalg.block_diag(*([eW2] * 8)) * (1.0 / 1000.0)
    b1rep = jnp.tile(eb1, 8).reshape(1, 128)
    b2rep = jnp.tile(eb2, 8).reshape(1, 128) * (1.0 / 1000.0)
    zeros_rpt = jnp.zeros((RPT, H), jnp.float32)

    h = _node_mlp(xt, bd, bcat, mW1, mb1.reshape(1, H), mW2,
                  mb2.reshape(1, H))
    sq1 = _dist1(posp, edge_index)
    sq2 = _dist2(posp, edge_index)
    e1 = _edge_mlp(eat, sq1, eW1a, w1bd, b1rep, w2bd, b2rep, 12, 0)
    e2 = _edge_mlp(eat, sq2, eW1a, w1bd, b1rep, w2bd, b2rep, 13, 12)
    agg1 = _scat1(h, e1, edge_index, dst2d, zeros_rpt)
    agg2 = _scat2(h, e2, edge_index, dst2d, zeros_rpt)
    return _final_kernel(h, agg1, agg2)
